# baseline (speedup 1.0000x reference)
import functools

import jax
import jax.numpy as jnp
from jax import lax
from jax.experimental import pallas as pl
from jax.experimental.pallas import tpu as pltpu
from jax.experimental.pallas import tpu_sc as plsc

E = 8
D = 1024
T = 4096
C = 512
TB = 256
NB = T // TB

NC, NS = 2, 16
NW = NC * NS
SPW = T // NW
RCH = 32
NCH = SPW // RCH
RB = 4
NH = D // RB


def _router_body(x_ref, wr_ref, br_ref, slot_ref, gate_ref, carry_ref, tri_ref):
    b = pl.program_id(0)

    @pl.when(b == 0)
    def _init():
        carry_ref[...] = jnp.zeros_like(carry_ref)
        ri = lax.broadcasted_iota(jnp.int32, (TB, TB), 0)
        ci = lax.broadcasted_iota(jnp.int32, (TB, TB), 1)
        tri_ref[...] = (ci <= ri).astype(jnp.float32)

    xb = x_ref[...]
    logits = (
        jnp.dot(xb, wr_ref[...], preferred_element_type=jnp.float32)
        + br_ref[...]
    )
    lmax = jnp.max(logits, axis=1, keepdims=True)
    ex = jnp.exp(logits - lmax)
    probs = ex / jnp.sum(ex, axis=1, keepdims=True)
    gate = jnp.max(probs, axis=1)
    pmax = jnp.max(probs, axis=1, keepdims=True)
    lane = lax.broadcasted_iota(jnp.int32, (TB, E), 1)
    idx = jnp.min(jnp.where(probs == pmax, lane, E), axis=1)
    onehot = (lane == idx[:, None]).astype(jnp.float32)
    csum = jnp.dot(tri_ref[...], onehot, preferred_element_type=jnp.float32)
    pos = csum + carry_ref[...]
    carry_ref[...] = carry_ref[...] + csum[TB - 1 : TB, :]
    p = jnp.sum(pos * onehot, axis=1)
    kept = p < float(C)
    gate_m = gate * kept.astype(jnp.float32)
    slot = idx * C + jnp.where(kept, p.astype(jnp.int32), 0)
    slot_ref[0, 0, :] = slot
    gate_ref[0, 0, :] = gate_m


_router_call = pl.pallas_call(
    _router_body,
    grid=(NB,),
    in_specs=[
        pl.BlockSpec((TB, D), lambda b: (b, 0)),
        pl.BlockSpec((D, E), lambda b: (0, 0)),
        pl.BlockSpec((1, E), lambda b: (0, 0)),
    ],
    out_specs=[
        pl.BlockSpec((1, 1, TB), lambda b: (b, 0, 0)),
        pl.BlockSpec((1, 1, TB), lambda b: (b, 0, 0)),
    ],
    out_shape=[
        jax.ShapeDtypeStruct((NB, 1, TB), jnp.int32),
        jax.ShapeDtypeStruct((NB, 1, TB), jnp.float32),
    ],
    scratch_shapes=[pltpu.VMEM((1, E), jnp.float32),
                    pltpu.VMEM((TB, TB), jnp.float32)],
)


NBUF = 3


def _gathered_rows_out(src_hbm, idx_ref, dst_hbm, base, bufs, gsems, osems):
    gps = [None] * NCH
    ops = [None] * NCH
    for c in range(min(2, NCH)):
        gps[c] = pltpu.async_copy(
            src_hbm.at[idx_ref.at[pl.ds(c * RCH, RCH)]],
            bufs[c % NBUF], gsems[c % NBUF])
    for c in range(NCH):
        gps[c].wait()
        ops[c] = pltpu.async_copy(
            bufs[c % NBUF], dst_hbm.at[pl.ds(base + c * RCH, RCH)],
            osems[c % NBUF])
        n = c + 2
        if n < NCH:
            if c >= 1:
                ops[c - 1].wait()
            gps[n] = pltpu.async_copy(
                src_hbm.at[idx_ref.at[pl.ds(n * RCH, RCH)]],
                bufs[n % NBUF], gsems[n % NBUF])
    for c in range(max(0, NCH - 2), NCH):
        ops[c].wait()


def _dispatch_body(x_hbm, slot_hbm, gate_hbm, ei_hbm, gslot_hbm,
                   slot_v, gate_v, tok_win, gate_win,
                   buf0, buf1, buf2, gs0, gs1, gs2, os0, os1, os2):
    wid = lax.axis_index("s") * NC + lax.axis_index("c")
    base = wid * SPW
    pltpu.sync_copy(slot_hbm, slot_v)
    pltpu.sync_copy(gate_hbm, gate_v)
    for i in range(SPW // 16):
        tok_win[pl.ds(i * 16, 16)] = jnp.zeros((16,), jnp.int32)
        gate_win[pl.ds(i * 16, 16)] = jnp.zeros((16,), jnp.float32)
    lane = lax.iota(jnp.int32, 16)

    def scatter_step(i, _):
        for u in range(2):
            off = i * 32 + u * 16
            s16 = slot_v[pl.ds(off, 16)]
            g16 = gate_v[pl.ds(off, 16)]
            t16 = lane + off
            loc = s16 - base
            m = (loc >= 0) & (loc < SPW)
            locc = jnp.clip(loc, 0, SPW - 1)
            plsc.store_scatter(tok_win, [locc], t16, mask=m)
            plsc.store_scatter(gate_win, [locc], g16, mask=m)
        return 0

    lax.fori_loop(0, T // 32, scatter_step, 0)
    pltpu.sync_copy(gate_win, gslot_hbm.at[pl.ds(base, SPW)])

    _gathered_rows_out(x_hbm, tok_win, ei_hbm, base,
                       (buf0, buf1, buf2), (gs0, gs1, gs2), (os0, os1, os2))


def _ffn_body(ei_ref, w1_ref, b1_ref, w2_ref, b2_ref, g_ref, y_ref):
    g = g_ref[0, 0][:, None]
    xb = ei_ref[0] * g
    h = jnp.dot(xb, w1_ref[0], preferred_element_type=jnp.float32)
    h = jax.nn.relu(h + b1_ref[0])
    y = jnp.dot(h, w2_ref[0], preferred_element_type=jnp.float32)
    y_ref[0] = (y + b2_ref[0]) * g


_ffn_call = pl.pallas_call(
    _ffn_body,
    grid=(E,),
    in_specs=[
        pl.BlockSpec((1, C, D), lambda e: (e, 0, 0)),
        pl.BlockSpec((1, D, D), lambda e: (e, 0, 0)),
        pl.BlockSpec((1, 1, D), lambda e: (e, 0, 0)),
        pl.BlockSpec((1, D, D), lambda e: (e, 0, 0)),
        pl.BlockSpec((1, 1, D), lambda e: (e, 0, 0)),
        pl.BlockSpec((1, 1, C), lambda e: (e, 0, 0)),
    ],
    out_specs=pl.BlockSpec((1, C, D), lambda e: (e, 0, 0)),
    out_shape=jax.ShapeDtypeStruct((E, C, D), jnp.float32),
    compiler_params=pltpu.CompilerParams(vmem_limit_bytes=100 * 1024 * 1024),
)


def _combine_body(y_hbm, slot_hbm, out_hbm, idx_v,
                  buf0, buf1, buf2, gs0, gs1, gs2, os0, os1, os2):
    wid = lax.axis_index("s") * NC + lax.axis_index("c")
    base = wid * SPW
    pltpu.sync_copy(slot_hbm.at[pl.ds(base, SPW)], idx_v)
    _gathered_rows_out(y_hbm, idx_v, out_hbm, base,
                       (buf0, buf1, buf2), (gs0, gs1, gs2), (os0, os1, os2))


@functools.lru_cache(maxsize=None)
def _sc_calls():
    mesh = plsc.VectorSubcoreMesh(
        core_axis_name="c", subcore_axis_name="s",
        num_cores=NC, num_subcores=NS)
    dispatch = pl.kernel(
        _dispatch_body,
        out_type=(
            jax.ShapeDtypeStruct((T, D), jnp.float32),
            jax.ShapeDtypeStruct((T,), jnp.float32),
        ),
        mesh=mesh,
        compiler_params=pltpu.CompilerParams(needs_layout_passes=False),
        scratch_types=[
            pltpu.VMEM((T,), jnp.int32),
            pltpu.VMEM((T,), jnp.float32),
            pltpu.VMEM((SPW,), jnp.int32),
            pltpu.VMEM((SPW,), jnp.float32),
            pltpu.VMEM((RCH, D), jnp.float32),
            pltpu.VMEM((RCH, D), jnp.float32),
            pltpu.VMEM((RCH, D), jnp.float32),
            pltpu.SemaphoreType.DMA,
            pltpu.SemaphoreType.DMA,
            pltpu.SemaphoreType.DMA,
            pltpu.SemaphoreType.DMA,
            pltpu.SemaphoreType.DMA,
            pltpu.SemaphoreType.DMA,
        ],
    )
    combine = pl.kernel(
        _combine_body,
        out_type=jax.ShapeDtypeStruct((T, D), jnp.float32),
        mesh=mesh,
        compiler_params=pltpu.CompilerParams(needs_layout_passes=False),
        scratch_types=[
            pltpu.VMEM((SPW,), jnp.int32),
            pltpu.VMEM((RCH, D), jnp.float32),
            pltpu.VMEM((RCH, D), jnp.float32),
            pltpu.VMEM((RCH, D), jnp.float32),
            pltpu.SemaphoreType.DMA,
            pltpu.SemaphoreType.DMA,
            pltpu.SemaphoreType.DMA,
            pltpu.SemaphoreType.DMA,
            pltpu.SemaphoreType.DMA,
            pltpu.SemaphoreType.DMA,
        ],
    )
    return dispatch, combine


def kernel(inputs, Wr, br, W1, b1, W2, b2):
    x = inputs.reshape(T, D)
    slot3, gate3 = _router_call(x, Wr, br.reshape(1, E))
    slot = slot3.reshape(T)
    gate = gate3.reshape(T)
    dispatch, combine = _sc_calls()
    ei, gslot = dispatch(x, slot, gate)
    y = _ffn_call(ei.reshape(E, C, D), W1, b1.reshape(E, 1, D),
                  W2, b2.reshape(E, 1, D), gslot.reshape(E, 1, C))
    out = combine(y.reshape(T, D), slot)
    return out.reshape(inputs.shape)


# speedup vs baseline: 1.0482x; 1.0482x over previous
import functools

import jax
import jax.numpy as jnp
from jax import lax
from jax.experimental import pallas as pl
from jax.experimental.pallas import tpu as pltpu
from jax.experimental.pallas import tpu_sc as plsc

E = 8
D = 1024
T = 4096
C = 512
TB = 512
NB = T // TB

NC, NS = 2, 16
NW = NC * NS
SPW = T // NW
RCH = 32
NCH = SPW // RCH
RB = 4
NH = D // RB


def _router_body(x_ref, wr_ref, br_ref, slot_ref, gate_ref, carry_ref, tri_ref):
    b = pl.program_id(0)

    @pl.when(b == 0)
    def _init():
        carry_ref[...] = jnp.zeros_like(carry_ref)
        ri = lax.broadcasted_iota(jnp.int32, (TB, TB), 0)
        ci = lax.broadcasted_iota(jnp.int32, (TB, TB), 1)
        tri_ref[...] = (ci <= ri).astype(jnp.float32)

    xb = x_ref[...]
    logits = (
        jnp.dot(xb, wr_ref[...], preferred_element_type=jnp.float32)
        + br_ref[...]
    )
    lmax = jnp.max(logits, axis=1, keepdims=True)
    ex = jnp.exp(logits - lmax)
    probs = ex / jnp.sum(ex, axis=1, keepdims=True)
    gate = jnp.max(probs, axis=1)
    pmax = jnp.max(probs, axis=1, keepdims=True)
    lane = lax.broadcasted_iota(jnp.int32, (TB, E), 1)
    idx = jnp.min(jnp.where(probs == pmax, lane, E), axis=1)
    onehot = (lane == idx[:, None]).astype(jnp.float32)
    csum = jnp.dot(tri_ref[...], onehot, preferred_element_type=jnp.float32)
    pos = csum + carry_ref[...]
    carry_ref[...] = carry_ref[...] + csum[TB - 1 : TB, :]
    p = jnp.sum(pos * onehot, axis=1)
    kept = p < float(C)
    gate_m = gate * kept.astype(jnp.float32)
    slot = idx * C + jnp.where(kept, p.astype(jnp.int32), 0)
    slot_ref[0, 0, :] = slot
    gate_ref[0, 0, :] = gate_m


_router_call = pl.pallas_call(
    _router_body,
    grid=(NB,),
    in_specs=[
        pl.BlockSpec((TB, D), lambda b: (b, 0)),
        pl.BlockSpec((D, E), lambda b: (0, 0)),
        pl.BlockSpec((1, E), lambda b: (0, 0)),
    ],
    out_specs=[
        pl.BlockSpec((1, 1, TB), lambda b: (b, 0, 0)),
        pl.BlockSpec((1, 1, TB), lambda b: (b, 0, 0)),
    ],
    out_shape=[
        jax.ShapeDtypeStruct((NB, 1, TB), jnp.int32),
        jax.ShapeDtypeStruct((NB, 1, TB), jnp.float32),
    ],
    scratch_shapes=[pltpu.VMEM((1, E), jnp.float32),
                    pltpu.VMEM((TB, TB), jnp.float32)],
)


NBUF = 3


def _gathered_rows_out(src_hbm, idx_ref, dst_hbm, base, bufs, gsems, osems):
    gps = [None] * NCH
    ops = [None] * NCH
    for c in range(min(2, NCH)):
        gps[c] = pltpu.async_copy(
            src_hbm.at[idx_ref.at[pl.ds(c * RCH, RCH)]],
            bufs[c % NBUF], gsems[c % NBUF])
    for c in range(NCH):
        gps[c].wait()
        ops[c] = pltpu.async_copy(
            bufs[c % NBUF], dst_hbm.at[pl.ds(base + c * RCH, RCH)],
            osems[c % NBUF])
        n = c + 2
        if n < NCH:
            if c >= 1:
                ops[c - 1].wait()
            gps[n] = pltpu.async_copy(
                src_hbm.at[idx_ref.at[pl.ds(n * RCH, RCH)]],
                bufs[n % NBUF], gsems[n % NBUF])
    for c in range(max(0, NCH - 2), NCH):
        ops[c].wait()


def _dispatch_body(x_hbm, slot_hbm, gate_hbm, ei_hbm, gslot_hbm,
                   slot_v, gate_v, tok_win, gate_win,
                   buf0, buf1, buf2, gs0, gs1, gs2, os0, os1, os2):
    wid = lax.axis_index("s") * NC + lax.axis_index("c")
    base = wid * SPW
    pltpu.sync_copy(slot_hbm, slot_v)
    pltpu.sync_copy(gate_hbm, gate_v.at[pl.ds(0, T)])
    for i in range(16 // 16):
        gate_v[pl.ds(T + i * 16, 16)] = jnp.zeros((16,), jnp.float32)
    for i in range(SPW // 16):
        tok_win[pl.ds(i * 16, 16)] = jnp.full((16,), T, jnp.int32)
    lane = lax.iota(jnp.int32, 16)

    def scatter_step(i, _):
        for u in range(2):
            off = i * 32 + u * 16
            s16 = slot_v[pl.ds(off, 16)]
            t16 = lane + off
            loc = s16 - base
            m = (loc >= 0) & (loc < SPW)
            locc = jnp.clip(loc, 0, SPW - 1)
            plsc.store_scatter(tok_win, [locc], t16, mask=m)
        return 0

    lax.fori_loop(0, T // 32, scatter_step, 0)
    for i in range(SPW // 16):
        t16 = tok_win[pl.ds(i * 16, 16)]
        gate_win[pl.ds(i * 16, 16)] = plsc.load_gather(gate_v, [t16])
    pltpu.sync_copy(gate_win, gslot_hbm.at[pl.ds(base, SPW)])

    _gathered_rows_out(x_hbm, tok_win, ei_hbm, base,
                       (buf0, buf1, buf2), (gs0, gs1, gs2), (os0, os1, os2))


def _ffn_body(ei_ref, w1_ref, b1_ref, w2_ref, b2_ref, g_ref, y_ref):
    g = g_ref[0, 0][:, None]
    xb = ei_ref[0] * g
    h = jnp.dot(xb, w1_ref[0], preferred_element_type=jnp.float32)
    h = jax.nn.relu(h + b1_ref[0])
    y = jnp.dot(h, w2_ref[0], preferred_element_type=jnp.float32)
    y_ref[0] = (y + b2_ref[0]) * g


_ffn_call = pl.pallas_call(
    _ffn_body,
    grid=(E,),
    in_specs=[
        pl.BlockSpec((1, C, D), lambda e: (e, 0, 0)),
        pl.BlockSpec((1, D, D), lambda e: (e, 0, 0)),
        pl.BlockSpec((1, 1, D), lambda e: (e, 0, 0)),
        pl.BlockSpec((1, D, D), lambda e: (e, 0, 0)),
        pl.BlockSpec((1, 1, D), lambda e: (e, 0, 0)),
        pl.BlockSpec((1, 1, C), lambda e: (e, 0, 0)),
    ],
    out_specs=pl.BlockSpec((1, C, D), lambda e: (e, 0, 0)),
    out_shape=jax.ShapeDtypeStruct((E, C, D), jnp.float32),
    compiler_params=pltpu.CompilerParams(vmem_limit_bytes=100 * 1024 * 1024),
)


def _combine_body(y_hbm, slot_hbm, out_hbm, idx_v,
                  buf0, buf1, buf2, gs0, gs1, gs2, os0, os1, os2):
    wid = lax.axis_index("s") * NC + lax.axis_index("c")
    base = wid * SPW
    pltpu.sync_copy(slot_hbm.at[pl.ds(base, SPW)], idx_v)
    _gathered_rows_out(y_hbm, idx_v, out_hbm, base,
                       (buf0, buf1, buf2), (gs0, gs1, gs2), (os0, os1, os2))


@functools.lru_cache(maxsize=None)
def _sc_calls():
    mesh = plsc.VectorSubcoreMesh(
        core_axis_name="c", subcore_axis_name="s",
        num_cores=NC, num_subcores=NS)
    dispatch = pl.kernel(
        _dispatch_body,
        out_type=(
            jax.ShapeDtypeStruct((T, D), jnp.float32),
            jax.ShapeDtypeStruct((T,), jnp.float32),
        ),
        mesh=mesh,
        compiler_params=pltpu.CompilerParams(needs_layout_passes=False),
        scratch_types=[
            pltpu.VMEM((T,), jnp.int32),
            pltpu.VMEM((T + 16,), jnp.float32),
            pltpu.VMEM((SPW,), jnp.int32),
            pltpu.VMEM((SPW,), jnp.float32),
            pltpu.VMEM((RCH, D), jnp.float32),
            pltpu.VMEM((RCH, D), jnp.float32),
            pltpu.VMEM((RCH, D), jnp.float32),
            pltpu.SemaphoreType.DMA,
            pltpu.SemaphoreType.DMA,
            pltpu.SemaphoreType.DMA,
            pltpu.SemaphoreType.DMA,
            pltpu.SemaphoreType.DMA,
            pltpu.SemaphoreType.DMA,
        ],
    )
    combine = pl.kernel(
        _combine_body,
        out_type=jax.ShapeDtypeStruct((T, D), jnp.float32),
        mesh=mesh,
        compiler_params=pltpu.CompilerParams(needs_layout_passes=False),
        scratch_types=[
            pltpu.VMEM((SPW,), jnp.int32),
            pltpu.VMEM((RCH, D), jnp.float32),
            pltpu.VMEM((RCH, D), jnp.float32),
            pltpu.VMEM((RCH, D), jnp.float32),
            pltpu.SemaphoreType.DMA,
            pltpu.SemaphoreType.DMA,
            pltpu.SemaphoreType.DMA,
            pltpu.SemaphoreType.DMA,
            pltpu.SemaphoreType.DMA,
            pltpu.SemaphoreType.DMA,
        ],
    )
    return dispatch, combine


def kernel(inputs, Wr, br, W1, b1, W2, b2):
    x = inputs.reshape(T, D)
    slot3, gate3 = _router_call(x, Wr, br.reshape(1, E))
    slot = slot3.reshape(T)
    gate = gate3.reshape(T)
    dispatch, combine = _sc_calls()
    ei, gslot = dispatch(x, slot, gate)
    y = _ffn_call(ei.reshape(E, C, D), W1, b1.reshape(E, 1, D),
                  W2, b2.reshape(E, 1, D), gslot.reshape(E, 1, C))
    out = combine(y.reshape(T, D), slot)
    return out.reshape(inputs.shape)


# speedup vs baseline: 1.0516x; 1.0032x over previous
import functools

import jax
import jax.numpy as jnp
from jax import lax
from jax.experimental import pallas as pl
from jax.experimental.pallas import tpu as pltpu
from jax.experimental.pallas import tpu_sc as plsc

E = 8
D = 1024
T = 4096
C = 512
TB = 512
NB = T // TB

NC, NS = 2, 16
NW = NC * NS
SPW = T // NW
RCH = 32
NCH = SPW // RCH
NBUF = 3


def _router_body(x_ref, wr_ref, br_ref, slot_ref, gate_ref, carry_ref, tri_ref):
    b = pl.program_id(0)

    @pl.when(b == 0)
    def _init():
        carry_ref[...] = jnp.zeros_like(carry_ref)
        ri = lax.broadcasted_iota(jnp.int32, (TB, TB), 0)
        ci = lax.broadcasted_iota(jnp.int32, (TB, TB), 1)
        tri_ref[...] = (ci <= ri).astype(jnp.float32)

    xb = x_ref[...]
    logits = (
        jnp.dot(xb, wr_ref[...], preferred_element_type=jnp.float32)
        + br_ref[...]
    )
    lmax = jnp.max(logits, axis=1, keepdims=True)
    ex = jnp.exp(logits - lmax)
    probs = ex / jnp.sum(ex, axis=1, keepdims=True)
    gate = jnp.max(probs, axis=1)
    pmax = jnp.max(probs, axis=1, keepdims=True)
    lane = lax.broadcasted_iota(jnp.int32, (TB, E), 1)
    idx = jnp.min(jnp.where(probs == pmax, lane, E), axis=1)
    onehot = (lane == idx[:, None]).astype(jnp.float32)
    csum = jnp.dot(tri_ref[...], onehot, preferred_element_type=jnp.float32)
    pos = csum + carry_ref[...]
    carry_ref[...] = carry_ref[...] + csum[TB - 1 : TB, :]
    p = jnp.sum(pos * onehot, axis=1)
    kept = p < float(C)
    gate_m = gate * kept.astype(jnp.float32)
    slot = idx * C + jnp.where(kept, p.astype(jnp.int32), 0)
    slot_ref[0, 0, :] = slot
    gate_ref[0, 0, :] = gate_m


_router_call = pl.pallas_call(
    _router_body,
    grid=(NB,),
    in_specs=[
        pl.BlockSpec((TB, D), lambda b: (b, 0)),
        pl.BlockSpec((D, E), lambda b: (0, 0)),
        pl.BlockSpec((1, E), lambda b: (0, 0)),
    ],
    out_specs=[
        pl.BlockSpec((1, 1, TB), lambda b: (b, 0, 0)),
        pl.BlockSpec((1, 1, TB), lambda b: (b, 0, 0)),
    ],
    out_shape=[
        jax.ShapeDtypeStruct((NB, 1, TB), jnp.int32),
        jax.ShapeDtypeStruct((NB, 1, TB), jnp.float32),
    ],
    scratch_shapes=[pltpu.VMEM((1, E), jnp.float32),
                    pltpu.VMEM((TB, TB), jnp.float32)],
)


def _gathered_rows_out(src_hbm, idx_ref, dst_hbm, base, bufs, gsems, osems):
    gps = [None] * NCH
    ops = [None] * NCH
    for c in range(min(2, NCH)):
        gps[c] = pltpu.async_copy(
            src_hbm.at[idx_ref.at[pl.ds(c * RCH, RCH)]],
            bufs[c % NBUF], gsems[c % NBUF])
    for c in range(NCH):
        gps[c].wait()
        ops[c] = pltpu.async_copy(
            bufs[c % NBUF], dst_hbm.at[pl.ds(base + c * RCH, RCH)],
            osems[c % NBUF])
        n = c + 2
        if n < NCH:
            if c >= 1:
                ops[c - 1].wait()
            gps[n] = pltpu.async_copy(
                src_hbm.at[idx_ref.at[pl.ds(n * RCH, RCH)]],
                bufs[n % NBUF], gsems[n % NBUF])
    for c in range(max(0, NCH - 2), NCH):
        ops[c].wait()


def _dispatch_body(x_hbm, slot_hbm, gate_hbm, ei_hbm, gslot_hbm,
                   slot_v, gate_v, tok_win, gate_win,
                   buf0, buf1, buf2, gs0, gs1, gs2, os0, os1, os2):
    wid = lax.axis_index("s") * NC + lax.axis_index("c")
    base = wid * SPW
    pltpu.sync_copy(slot_hbm, slot_v)
    pltpu.sync_copy(gate_hbm, gate_v.at[pl.ds(0, T)])
    gate_v[pl.ds(T, 16)] = jnp.zeros((16,), jnp.float32)
    for i in range(SPW // 16):
        tok_win[pl.ds(i * 16, 16)] = jnp.full((16,), T, jnp.int32)
    lane = lax.iota(jnp.int32, 16)

    def scatter_step(i, _):
        for u in range(2):
            off = i * 32 + u * 16
            s16 = slot_v[pl.ds(off, 16)]
            t16 = lane + off
            loc = s16 - base
            m = (loc >= 0) & (loc < SPW)
            locc = jnp.clip(loc, 0, SPW - 1)
            plsc.store_scatter(tok_win, [locc], t16, mask=m)
        return 0

    lax.fori_loop(0, T // 32, scatter_step, 0)
    for i in range(SPW // 16):
        t16 = tok_win[pl.ds(i * 16, 16)]
        gate_win[pl.ds(i * 16, 16)] = plsc.load_gather(gate_v, [t16])
    pltpu.sync_copy(gate_win, gslot_hbm.at[pl.ds(base, SPW)])
    _gathered_rows_out(x_hbm, tok_win, ei_hbm, base,
                       (buf0, buf1, buf2), (gs0, gs1, gs2), (os0, os1, os2))


def _ffn_body(ei_ref, w1_ref, b1_ref, w2_ref, b2_ref, g_ref, y_ref):
    g = g_ref[0, 0][:, None]
    xb = ei_ref[0] * g
    h = jnp.dot(xb, w1_ref[0], preferred_element_type=jnp.float32)
    h = jax.nn.relu(h + b1_ref[0])
    y = jnp.dot(h, w2_ref[0], preferred_element_type=jnp.float32)
    y_ref[0] = (y + b2_ref[0]) * g


_ffn_call = pl.pallas_call(
    _ffn_body,
    grid=(E,),
    in_specs=[
        pl.BlockSpec((1, C, D), lambda e: (e, 0, 0)),
        pl.BlockSpec((1, D, D), lambda e: (e, 0, 0)),
        pl.BlockSpec((1, 1, D), lambda e: (e, 0, 0)),
        pl.BlockSpec((1, D, D), lambda e: (e, 0, 0)),
        pl.BlockSpec((1, 1, D), lambda e: (e, 0, 0)),
        pl.BlockSpec((1, 1, C), lambda e: (e, 0, 0)),
    ],
    out_specs=pl.BlockSpec((1, C, D), lambda e: (e, 0, 0)),
    out_shape=jax.ShapeDtypeStruct((E, C, D), jnp.float32),
    compiler_params=pltpu.CompilerParams(vmem_limit_bytes=100 * 1024 * 1024),
)


def _combine_body(y_hbm, slot_hbm, out_hbm, idx_v,
                  buf0, buf1, buf2, gs0, gs1, gs2, os0, os1, os2):
    wid = lax.axis_index("s") * NC + lax.axis_index("c")
    base = wid * SPW
    pltpu.sync_copy(slot_hbm.at[pl.ds(base, SPW)], idx_v)
    _gathered_rows_out(y_hbm, idx_v, out_hbm, base,
                       (buf0, buf1, buf2), (gs0, gs1, gs2), (os0, os1, os2))


@functools.lru_cache(maxsize=None)
def _sc_calls():
    mesh = plsc.VectorSubcoreMesh(
        core_axis_name="c", subcore_axis_name="s",
        num_cores=NC, num_subcores=NS)
    dispatch = pl.kernel(
        _dispatch_body,
        out_type=(
            jax.ShapeDtypeStruct((T, D), jnp.float32),
            jax.ShapeDtypeStruct((T,), jnp.float32),
        ),
        mesh=mesh,
        compiler_params=pltpu.CompilerParams(needs_layout_passes=False),
        scratch_types=[
            pltpu.VMEM((T,), jnp.int32),
            pltpu.VMEM((T + 16,), jnp.float32),
            pltpu.VMEM((SPW,), jnp.int32),
            pltpu.VMEM((SPW,), jnp.float32),
            pltpu.VMEM((RCH, D), jnp.float32),
            pltpu.VMEM((RCH, D), jnp.float32),
            pltpu.VMEM((RCH, D), jnp.float32),
            pltpu.SemaphoreType.DMA,
            pltpu.SemaphoreType.DMA,
            pltpu.SemaphoreType.DMA,
            pltpu.SemaphoreType.DMA,
            pltpu.SemaphoreType.DMA,
            pltpu.SemaphoreType.DMA,
        ],
    )
    combine = pl.kernel(
        _combine_body,
        out_type=jax.ShapeDtypeStruct((T, D), jnp.float32),
        mesh=mesh,
        compiler_params=pltpu.CompilerParams(needs_layout_passes=False),
        scratch_types=[
            pltpu.VMEM((SPW,), jnp.int32),
            pltpu.VMEM((RCH, D), jnp.float32),
            pltpu.VMEM((RCH, D), jnp.float32),
            pltpu.VMEM((RCH, D), jnp.float32),
            pltpu.SemaphoreType.DMA,
            pltpu.SemaphoreType.DMA,
            pltpu.SemaphoreType.DMA,
            pltpu.SemaphoreType.DMA,
            pltpu.SemaphoreType.DMA,
            pltpu.SemaphoreType.DMA,
        ],
    )
    return dispatch, combine


def kernel(inputs, Wr, br, W1, b1, W2, b2):
    x = inputs.reshape(T, D)
    slot3, gate3 = _router_call(x, Wr, br.reshape(1, E))
    slot = slot3.reshape(T)
    gate = gate3.reshape(T)
    dispatch, combine = _sc_calls()
    ei, gslot = dispatch(x, slot, gate)
    y = _ffn_call(ei.reshape(E, C, D), W1, b1.reshape(E, 1, D),
                  W2, b2.reshape(E, 1, D), gslot.reshape(E, 1, C))
    out = combine(y.reshape(T, D), slot)
    return out.reshape(inputs.shape)
